# Initial kernel scaffold; baseline (speedup 1.0000x reference)
import jax
import jax.numpy as jnp
from jax import lax
from jax.experimental import pallas as pl
from jax.experimental.pallas import tpu as pltpu
from jax.experimental.pallas import tpu_sc as plsc

N = 10000
E = 320000
F = 128
C = 128

NC = 2
NS = 16
NW = NC * NS
EPW = E // NW
K = 80
NCHUNK = EPW // K
RPT = N // NS
ZR = 125


def _mm_body(x_ref, w_ref, o_ref):
    o_ref[...] = jnp.dot(x_ref[...], w_ref[...],
                         preferred_element_type=jnp.float32)


def _matmul(x, W):
    RB = 2000
    return pl.pallas_call(
        _mm_body,
        grid=(N // RB,),
        in_specs=[pl.BlockSpec((RB, F), lambda i: (i, 0)),
                  pl.BlockSpec((F, C), lambda i: (0, 0))],
        out_specs=pl.BlockSpec((RB, C), lambda i: (i, 0)),
        out_shape=jax.ShapeDtypeStruct((N, C), jnp.float32),
    )(x, W)


def _comb_body(p_ref, b_ref, o_ref):
    o_ref[...] = p_ref[0] + p_ref[1] + b_ref[...]


def _combine(p, b):
    RB = 2000
    return pl.pallas_call(
        _comb_body,
        grid=(N // RB,),
        in_specs=[pl.BlockSpec((2, RB, C), lambda i: (0, i, 0)),
                  pl.BlockSpec((1, C), lambda i: (0, 0))],
        out_specs=pl.BlockSpec((RB, C), lambda i: (i, 0)),
        out_shape=jax.ShapeDtypeStruct((N, C), jnp.float32),
    )(p, b.reshape(1, C))


def _sc_body(h_hbm, src_hbm, dst_hbm, w_hbm, out_hbm,
             src_v, dst_v, w_v, rows_v, zb_v, acc_sh, sem):
    c = lax.axis_index("c")
    s = lax.axis_index("s")
    wid = c * NS + s

    pltpu.sync_copy(src_hbm.at[wid], src_v)
    pltpu.sync_copy(dst_hbm.at[wid], dst_v)
    pltpu.sync_copy(w_hbm.at[wid], w_v)

    zero = jnp.zeros((16,), jnp.float32)

    def zrow(i, carry):
        for j in range(C // 16):
            zb_v[i, pl.ds(j * 16, 16)] = zero
        return carry

    lax.fori_loop(0, ZR, zrow, 0)
    base = s * RPT
    for t in range(RPT // ZR):
        pltpu.sync_copy(zb_v, acc_sh.at[pl.ds(base + t * ZR, ZR)])
    plsc.subcore_barrier()

    def chunk(g, carry):
        pltpu.async_copy(h_hbm.at[src_v.at[g]], rows_v, sem).wait()

        def scale(i, carry2):
            idx = jnp.full((16,), g * K + i, jnp.int32)
            wb = plsc.load_gather(w_v, [idx])
            for j in range(C // 16):
                sl = pl.ds(j * 16, 16)
                rows_v[i, sl] = rows_v[i, sl] * wb
            return carry2

        lax.fori_loop(0, K, scale, 0)
        pltpu.sync_copy(rows_v, acc_sh.at[dst_v.at[g]], add=True)
        return carry

    lax.fori_loop(0, NCHUNK, chunk, 0)
    plsc.subcore_barrier()

    for t in range(RPT // ZR):
        sl = pl.ds(base + t * ZR, ZR)
        pltpu.sync_copy(acc_sh.at[sl], out_hbm.at[c, sl])


_sc_call = pl.kernel(
    _sc_body,
    out_type=jax.ShapeDtypeStruct((NC, N, C), jnp.float32),
    mesh=plsc.VectorSubcoreMesh(core_axis_name="c", subcore_axis_name="s"),
    scratch_types=[
        pltpu.VMEM((NCHUNK, K), jnp.int32),
        pltpu.VMEM((NCHUNK, K), jnp.int32),
        pltpu.VMEM((EPW,), jnp.float32),
        pltpu.VMEM((K, C), jnp.float32),
        pltpu.VMEM((ZR, C), jnp.float32),
        pltpu.VMEM_SHARED((N, C), jnp.float32),
        pltpu.SemaphoreType.DMA,
    ],
)


def kernel(x, edge_index, edge_weight, W, b):
    h = _matmul(x, W)
    src = edge_index[1].astype(jnp.int32).reshape(NW, NCHUNK, K)
    dst = edge_index[0].astype(jnp.int32).reshape(NW, NCHUNK, K)
    w = edge_weight.reshape(NW, EPW)
    p = _sc_call(h, src, dst, w)
    return _combine(p, b)


# speedup vs baseline: 4.6898x; 4.6898x over previous
import jax
import jax.numpy as jnp
from jax import lax
from jax.experimental import pallas as pl
from jax.experimental.pallas import tpu as pltpu
from jax.experimental.pallas import tpu_sc as plsc

N = 10000
E = 320000
F = 128
C = 128

NC = 2
NS = 16
NW = NC * NS
EPW = E // NW
K = 80
NCHUNK = EPW // K
NA = 10240
RPT = NA // NS
ZR = 128


def _mm_body(x_ref, w_ref, o_ref):
    o_ref[...] = jnp.dot(x_ref[...], w_ref[...],
                         preferred_element_type=jnp.float32)


def _matmul(x, W):
    RB = 2000
    return pl.pallas_call(
        _mm_body,
        grid=(N // RB,),
        in_specs=[pl.BlockSpec((RB, F), lambda i: (i, 0)),
                  pl.BlockSpec((F, C), lambda i: (0, 0))],
        out_specs=pl.BlockSpec((RB, C), lambda i: (i, 0)),
        out_shape=jax.ShapeDtypeStruct((N, C), jnp.float32),
    )(x, W)


def _comb_body(p_ref, b_ref, o_ref):
    o_ref[...] = p_ref[0] + p_ref[1] + b_ref[...]


def _combine(p, b):
    RB = 2000
    return pl.pallas_call(
        _comb_body,
        grid=(N // RB,),
        in_specs=[pl.BlockSpec((2, RB, C), lambda i: (0, i, 0)),
                  pl.BlockSpec((1, C), lambda i: (0, 0))],
        out_specs=pl.BlockSpec((RB, C), lambda i: (i, 0)),
        out_shape=jax.ShapeDtypeStruct((N, C), jnp.float32),
    )(p, b.reshape(1, C))


def _sc_body(h_hbm, e_hbm, w_hbm, out_hbm, eb_v, wb_v, rows_v, acc_sh, sem):
    c = lax.axis_index("c")
    s = lax.axis_index("s")
    wid = c * NS + s

    zero = jnp.zeros((16,), jnp.float32)

    def zrow(i, carry):
        for j in range(C // 16):
            rows_v[i, pl.ds(j * 16, 16)] = zero
        return carry

    lax.fori_loop(0, K, zrow, 0)
    base = s * RPT
    for t in range(RPT // K):
        pltpu.sync_copy(rows_v, acc_sh.at[pl.ds(base + t * K, K)])
    plsc.subcore_barrier()

    def chunk(g, carry):
        pltpu.sync_copy(e_hbm.at[wid, g], eb_v)
        pltpu.sync_copy(w_hbm.at[wid, g], wb_v)
        pltpu.async_copy(h_hbm.at[eb_v.at[0]], rows_v, sem).wait()

        def scale(q, carry2):
            wv = wb_v[0, pl.ds(q * 16, 16)]
            for i in range(16):
                wb = jnp.full((16,), wv[i], jnp.float32)
                row = q * 16 + i
                for j in range(C // 16):
                    sl = pl.ds(j * 16, 16)
                    rows_v[row, sl] = rows_v[row, sl] * wb
            return carry2

        lax.fori_loop(0, K // 16, scale, 0)
        pltpu.sync_copy(rows_v, acc_sh.at[eb_v.at[1]], add=True)
        return carry

    lax.fori_loop(0, NCHUNK, chunk, 0)
    plsc.subcore_barrier()

    for t in range(RPT // ZR):
        sl = pl.ds(base + t * ZR, ZR)
        pltpu.sync_copy(acc_sh.at[sl], out_hbm.at[c, sl])


_sc_call = pl.kernel(
    _sc_body,
    out_type=jax.ShapeDtypeStruct((NC, NA, C), jnp.float32),
    mesh=plsc.VectorSubcoreMesh(core_axis_name="c", subcore_axis_name="s"),
    scratch_types=[
        pltpu.VMEM((2, K), jnp.int32),
        pltpu.VMEM((1, K), jnp.float32),
        pltpu.VMEM((K, C), jnp.float32),
        pltpu.VMEM_SHARED((NA, C), jnp.float32),
        pltpu.SemaphoreType.DMA,
    ],
)


def kernel(x, edge_index, edge_weight, W, b):
    h = _matmul(x, W)
    src = edge_index[1].astype(jnp.int32).reshape(NW, NCHUNK, 1, K)
    dst = edge_index[0].astype(jnp.int32).reshape(NW, NCHUNK, 1, K)
    e = jnp.concatenate([src, dst], axis=2)
    w = edge_weight.reshape(NW, NCHUNK, 1, K)
    p = _sc_call(h, e, w)
    return _combine(p, b)
